# baseline (speedup 1.0000x reference)
import functools

import jax
import jax.numpy as jnp
from jax import lax
from jax.experimental import pallas as pl
from jax.experimental.pallas import tpu as pltpu
from jax.experimental.pallas import tpu_sc as plsc

NC = 2
NS = 16
NW = NC * NS
C = 256
NBUF = 4
L = 16

D_MODEL = 64


def _make_gather(total, d):
    assert total % (NW * C) == 0
    b_per_w = total // NW
    nchunk = b_per_w // C
    assert nchunk % NBUF == 0
    mesh = plsc.VectorSubcoreMesh(core_axis_name="c", subcore_axis_name="s")

    @functools.partial(
        pl.kernel,
        mesh=mesh,
        compiler_params=pltpu.CompilerParams(
            use_tc_tiling_on_sc=False,
            disable_bounds_checks=True,
        ),
        out_type=jax.ShapeDtypeStruct((total, d), jnp.float32),
        scratch_types=[
            pltpu.VMEM((b_per_w,), jnp.int32),
            pltpu.VMEM((NBUF, C, d), jnp.float32),
            [pltpu.SemaphoreType.DMA] * NBUF,
        ],
    )
    def gather_kernel(idx_hbm, table_hbm, out_hbm, idx_v, rows_v, sems):
        cid = lax.axis_index("c")
        sid = lax.axis_index("s")
        wid = sid * NC + cid
        base = wid * b_per_w
        pltpu.sync_copy(idx_hbm.at[pl.ds(base, b_per_w)], idx_v)

        def fire(j, b):
            for k in range(C // L):
                iv = idx_v[pl.ds(j * C + k * L, L)]
                pltpu.async_copy(
                    table_hbm.at[iv], rows_v.at[b, pl.ds(k * L, L)], sems[b]
                )

        for b in range(NBUF):
            fire(b, b)

        def round_body(r, carry):
            j0 = r * NBUF
            for b in range(NBUF):
                j = j0 + b
                pltpu.make_async_copy(
                    table_hbm.at[idx_v[pl.ds(j * C, C)]], rows_v.at[b], sems[b]
                ).wait()
                pltpu.sync_copy(rows_v.at[b], out_hbm.at[pl.ds(base + j * C, C)])

                @pl.when(j + NBUF < nchunk)
                def _():
                    fire(j + NBUF, b)
            return carry

        lax.fori_loop(0, nchunk // NBUF, round_body, 0)

    return gather_kernel


def kernel(token_ids, weight):
    b, s = token_ids.shape
    d = weight.shape[1]
    total = b * s
    idx = token_ids.reshape(total).astype(jnp.int32)
    out = _make_gather(total, d)(idx, weight)
    return out.reshape(b, s, d)
